# Initial kernel scaffold; baseline (speedup 1.0000x reference)
import jax
import jax.numpy as jnp
from jax.experimental import pallas as pl


def kernel(prediction, logits):
    raise NotImplementedError("write your pallas kernel here")


# speedup vs baseline: 12.3418x; 12.3418x over previous
import jax
import jax.numpy as jnp
from jax.experimental import pallas as pl
from jax.experimental.pallas import tpu as pltpu

_CONF_THRES = 0.45
_IOU_THRES = 0.45
_MAX_DET = 300

_B = 4
_N = 20000
_ROWS = 8
_COLS = 2560
_NPAD = _ROWS * _COLS
_NEG_INF = float("-inf")


def _nms_kernel(data_ref, tab_ref, out_ref, d_ref):
    lin = (
        jax.lax.broadcasted_iota(jnp.int32, (_ROWS, _COLS), 0) * _COLS
        + jax.lax.broadcasted_iota(jnp.int32, (_ROWS, _COLS), 1)
    )

    cx = data_ref[:, 0]
    cy = data_ref[:, 1]
    w = data_ref[:, 2]
    h = data_ref[:, 3]
    obj = data_ref[:, 4]
    hw = w * 0.5
    hh = h * 0.5
    x1 = cx - hw
    y1 = cy - hh
    x2 = cx + hw
    y2 = cy + hh
    c0 = data_ref[:, 5] * obj
    c1 = data_ref[:, 6] * obj
    c2 = data_ref[:, 7] * obj
    conf = jnp.maximum(jnp.maximum(c0, c1), c2)
    valid = (obj > _CONF_THRES) & (conf > _CONF_THRES) & (lin[None] < _N)
    d_ref[:, 0] = x1
    d_ref[:, 1] = y1
    d_ref[:, 2] = x2
    d_ref[:, 3] = y2
    d_ref[:, 4] = (x2 - x1) * (y2 - y1)
    d_ref[:, 5] = jnp.where(valid, conf, _NEG_INF)

    def body(i, carry):
        for img in range(_B):
            s = d_ref[img, 5]
            m = jnp.max(s)
            ok = m > _NEG_INF
            cand = jnp.where(s == m, lin, jnp.int32(2**30))
            idx = jnp.min(cand)
            r = idx // _COLS
            c = idx - r * _COLS

            t = idx // 64
            rem = idx - t * 64
            su = rem // 8
            u = rem - su * 8
            chunk = tab_ref[img, pl.ds(t, 1)]
            chunk = pltpu.roll(chunk, (8 - su) % 8, axis=1)
            chunk = pltpu.roll(chunk, (128 - u * 16) % 128, axis=2)
            bcx = chunk[0, 0, 0]
            bcy = chunk[0, 0, 1]
            bw = chunk[0, 0, 2]
            bh = chunk[0, 0, 3]
            bobj = chunk[0, 0, 4]
            bc0 = chunk[0, 0, 5] * bobj
            bc1 = chunk[0, 0, 6] * bobj
            bc2 = chunk[0, 0, 7] * bobj
            bl0 = chunk[0, 0, 8]
            bl1 = chunk[0, 0, 9]
            bl2 = chunk[0, 0, 10]
            bhw = bw * 0.5
            bhh = bh * 0.5
            bx1 = bcx - bhw
            by1 = bcy - bhh
            bx2 = bcx + bhw
            by2 = bcy + bhh
            barea = (bx2 - bx1) * (by2 - by1)
            bm01 = jnp.maximum(bc0, bc1)
            bconf = jnp.maximum(bm01, bc2)
            bj = jnp.where(bc1 > bc0, 1.0, 0.0)
            bj = jnp.where(bc2 > bm01, 2.0, bj)

            ix1 = jnp.maximum(d_ref[img, 0], bx1)
            iy1 = jnp.maximum(d_ref[img, 1], by1)
            ix2 = jnp.minimum(d_ref[img, 2], bx2)
            iy2 = jnp.minimum(d_ref[img, 3], by2)
            inter = jnp.maximum(ix2 - ix1, 0.0) * jnp.maximum(iy2 - iy1, 0.0)
            iou = inter / (barea + d_ref[img, 4] - inter + 1e-9)
            supp = (iou > _IOU_THRES) | (lin == idx)
            d_ref[img, 5] = jnp.where(supp, _NEG_INF, s)

            mok = jnp.where(ok, 1.0, 0.0)
            zero = jnp.float32(0.0)
            vals = jnp.stack(
                [bx1, by1, bx2, by2, bconf, bj, bl0, bl1, bl2,
                 zero, zero, zero, zero, zero, zero, zero]
            ) * mok
            out_ref[img, pl.ds(i, 1)] = vals.reshape(1, 1, 16)
        return carry

    jax.lax.fori_loop(0, _MAX_DET, body, 0)


@jax.jit
def kernel(prediction, logits):
    pred_t = prediction.transpose(0, 2, 1)
    pred_t = jnp.pad(pred_t, ((0, 0), (0, 0), (0, _NPAD - _N)))
    data = pred_t.reshape(_B, 8, _ROWS, _COLS)

    raw = jnp.concatenate([prediction, logits], axis=-1)
    raw = jnp.pad(raw, ((0, 0), (0, _NPAD - _N), (0, 5)))
    tab = raw.reshape(_B, _NPAD // 64, 8, 8, 16).reshape(_B, _NPAD // 64, 8, 128)

    out = pl.pallas_call(
        _nms_kernel,
        out_shape=jax.ShapeDtypeStruct((_B, _MAX_DET, 1, 16), jnp.float32),
        scratch_shapes=[pltpu.VMEM((_B, 8, _ROWS, _COLS), jnp.float32)],
    )(data, tab)

    outt = out.reshape(_B, _MAX_DET, 16)
    return outt[:, :, 0:6], outt[:, :, 6:9]
